# Initial kernel scaffold; baseline (speedup 1.0000x reference)
import jax
import jax.numpy as jnp
from jax.experimental import pallas as pl


def kernel(mu, t, edge_index, Wm0, bm0, Wu0, bu0, Wm1, bm1, Wu1, bu1, Wm2, bm2, Wu2, bu2, Wm3, bm3, Wu3, bu3, We1, be1, We2, be2):
    raise NotImplementedError("write your pallas kernel here")


# speedup vs baseline: 6.8184x; 6.8184x over previous
import functools

import jax
import jax.numpy as jnp
from jax import lax
from jax.experimental import pallas as pl
from jax.experimental.pallas import tpu as pltpu
from jax.experimental.pallas import tpu_sc as plsc

HID = 64
NC, NS, L = 2, 16, 16
CHUNK = 512
KIDX = 128
F32 = jnp.float32
I32 = jnp.int32
HIGH = lax.Precision.HIGHEST


def _r16(x):
    return x


def _bdot(x, w):
    return jax.lax.dot(x, w, precision=HIGH, preferred_element_type=F32)


def _tc_prep0_body(h_ref, wm_ref, bm_ref, a_ref, c_ref):
    h = h_ref[:]
    wm = wm_ref[:]
    a_ref[:] = (_r16(h[:, 0:1]) * _r16(wm[0:1])
                + _r16(h[:, 1:2]) * _r16(wm[1:2]) + bm_ref[:])
    c_ref[:] = _r16(h[:, 0:1]) * _r16(wm[2:3]) + _r16(h[:, 1:2]) * _r16(wm[3:4])


def _tc_boundary_body(h_ref, agg_ref, wu_ref, bu_ref, wm_ref, bm_ref,
                      h_out, a_out, c_out, *, din):
    wu = wu_ref[:]
    if din == 2:
        h = h_ref[:]
        z = _r16(h[:, 0:1]) * _r16(wu[0:1]) + _r16(h[:, 1:2]) * _r16(wu[1:2])
    else:
        z = _bdot(h_ref[:], wu[:din])
    z = z + _bdot(agg_ref[:], wu[din:]) + bu_ref[:]
    hn = jax.nn.silu(z)
    h_out[:] = hn
    wm = wm_ref[:]
    a_out[:] = _bdot(hn, wm[:HID]) + bm_ref[:]
    c_out[:] = _bdot(hn, wm[HID:])


def _tc_rates_body(g_ref, w_ref, b_ref, r_ref):
    z = jnp.sum(_r16(g_ref[:]) * _r16(w_ref[:]), axis=1, keepdims=True) + b_ref[:]
    r_ref[:] = jnp.logaddexp(z, 0.0)


def _tc_final_body(rm_ref, out_ref, *, n):
    rm = rm_ref[:]
    rs = jnp.sum(rm, axis=2)
    ri = lax.broadcasted_iota(I32, (1, n, n), 1)
    ci = lax.broadcasted_iota(I32, (1, n, n), 2)
    out_ref[:] = jnp.where(ri == ci, -rs[:, :, None], rm)


def _tile_coords(b_per_sc, e_per_tile):
    c = lax.axis_index("c")
    s = lax.axis_index("s")
    tpb = NS // b_per_sc
    lb = s // tpb
    b = c * b_per_sc + lb
    e0 = (s % tpb) * e_per_tile
    return c, s, b, e0


def _build_idx(st_ref, idx_ref, off):
    for j in range(idx_ref.shape[0]):
        def body(i, _, j=j):
            idx_ref[j, pl.ds(i * L, L)] = st_ref[pl.ds(j * KIDX + i * L, L)] + off
            return 0
        lax.fori_loop(0, KIDX // L, body, 0)


def _silu_inplace(a_buf, c_buf):
    def body(r, _):
        for k in range(HID // L):
            v = a_buf[r, pl.ds(k * L, L)] + c_buf[r, pl.ds(k * L, L)]
            a_buf[r, pl.ds(k * L, L)] = v / (1.0 + jnp.exp(-v))
        return 0
    lax.fori_loop(0, CHUNK, body, 0)


def _sc_edge_body(src_hbm, dst_hbm, a_hbm, c_hbm, agg_hbm,
                  src_st, dst_st, idxs, idxd, a_buf, c_buf, agg_sh, sem,
                  *, n, e, b_total):
    b_per_sc = b_total // NC
    e_per_tile = (e * b_per_sc) // NS
    c, s, b, e0 = _tile_coords(b_per_sc, e_per_tile)
    rows_per_tile = (b_per_sc * n) // NS

    def zbody(i, _):
        a_buf[i // (HID // L), pl.ds((i % (HID // L)) * L, L)] = jnp.zeros((L,), F32)
        return 0
    lax.fori_loop(0, rows_per_tile * (HID // L), zbody, 0)
    zrow = c * b_per_sc * n + s * rows_per_tile
    pltpu.sync_copy(a_buf.at[pl.ds(0, rows_per_tile)], agg_sh.at[pl.ds(zrow, rows_per_tile)])
    plsc.subcore_barrier()

    def chunk(mc, _):
        base = e0 + mc * CHUNK
        pltpu.sync_copy(src_hbm.at[pl.ds(base, CHUNK)], src_st)
        pltpu.sync_copy(dst_hbm.at[pl.ds(base, CHUNK)], dst_st)
        _build_idx(src_st, idxs, b * n)
        _build_idx(dst_st, idxd, b * n)
        descs = []
        for j in range(CHUNK // KIDX):
            descs.append(pltpu.async_copy(
                a_hbm.at[idxs.at[j]], a_buf.at[pl.ds(j * KIDX, KIDX)], sem))
            descs.append(pltpu.async_copy(
                c_hbm.at[idxd.at[j]], c_buf.at[pl.ds(j * KIDX, KIDX)], sem))
        for d in descs:
            d.wait()
        _silu_inplace(a_buf, c_buf)
        for j in range(CHUNK // KIDX):
            pltpu.sync_copy(a_buf.at[pl.ds(j * KIDX, KIDX)],
                            agg_sh.at[idxd.at[j]], add=True)
        return 0

    lax.fori_loop(0, e_per_tile // CHUNK, chunk, 0)
    plsc.subcore_barrier()
    pltpu.sync_copy(agg_sh.at[pl.ds(zrow, rows_per_tile)],
                    agg_hbm.at[pl.ds(zrow, rows_per_tile)])


def _sc_readout_body(src_hbm, dst_hbm, p_hbm, q_hbm, g_hbm,
                     src_st, dst_st, idxs, idxd, a_buf, c_buf, sem,
                     *, n, e, b_total):
    b_per_sc = b_total // NC
    e_per_tile = (e * b_per_sc) // NS
    c, s, b, e0 = _tile_coords(b_per_sc, e_per_tile)

    def chunk(mc, _):
        base = e0 + mc * CHUNK
        pltpu.sync_copy(src_hbm.at[pl.ds(base, CHUNK)], src_st)
        pltpu.sync_copy(dst_hbm.at[pl.ds(base, CHUNK)], dst_st)
        _build_idx(src_st, idxs, b * n)
        _build_idx(dst_st, idxd, b * n)
        descs = []
        for j in range(CHUNK // KIDX):
            descs.append(pltpu.async_copy(
                p_hbm.at[idxs.at[j]], a_buf.at[pl.ds(j * KIDX, KIDX)], sem))
            descs.append(pltpu.async_copy(
                q_hbm.at[idxd.at[j]], c_buf.at[pl.ds(j * KIDX, KIDX)], sem))
        for d in descs:
            d.wait()
        _silu_inplace(a_buf, c_buf)
        pltpu.sync_copy(a_buf, g_hbm.at[pl.ds(b * e + base, CHUNK)])
        return 0

    lax.fori_loop(0, e_per_tile // CHUNK, chunk, 0)


def _sc_assembly_body(src_hbm, dst_hbm, rate_hbm, rm_hbm,
                      src_st, dst_st, rate_st, idx, zbuf,
                      *, n, e, b_total):
    b_per_sc = b_total // NC
    e_per_tile = (e * b_per_sc) // NS
    c, s, b, e0 = _tile_coords(b_per_sc, e_per_tile)

    zwords = zbuf.shape[0]
    def zbody(i, _):
        zbuf[pl.ds(i * L, L)] = jnp.zeros((L,), F32)
        return 0
    lax.fori_loop(0, zwords // L, zbody, 0)
    words_per_tile = (b_per_sc * n * n) // NS
    zstart = c * b_per_sc * n * n + s * words_per_tile
    def zcopy(z, _):
        pltpu.sync_copy(zbuf, rm_hbm.at[pl.ds(zstart + z * zwords, zwords)])
        return 0
    lax.fori_loop(0, words_per_tile // zwords, zcopy, 0)
    plsc.subcore_barrier()

    def chunk(mc, _):
        base = e0 + mc * CHUNK
        pltpu.sync_copy(src_hbm.at[pl.ds(base, CHUNK)], src_st)
        pltpu.sync_copy(dst_hbm.at[pl.ds(base, CHUNK)], dst_st)
        pltpu.sync_copy(rate_hbm.at[pl.ds(b * e + base, CHUNK)], rate_st)
        for j in range(CHUNK // KIDX):
            def body(i, _, j=j):
                vs = src_st[pl.ds(j * KIDX + i * L, L)]
                vd = dst_st[pl.ds(j * KIDX + i * L, L)]
                idx[j, pl.ds(i * L, L)] = vs * n + vd + b * n * n
                return 0
            lax.fori_loop(0, KIDX // L, body, 0)
        for j in range(CHUNK // KIDX):
            pltpu.sync_copy(rate_st.at[pl.ds(j * KIDX, KIDX)], rm_hbm.at[idx.at[j]])
        return 0

    lax.fori_loop(0, e_per_tile // CHUNK, chunk, 0)


def kernel(mu, t, edge_index, Wm0, bm0, Wu0, bu0, Wm1, bm1, Wu1, bu1,
           Wm2, bm2, Wu2, bu2, Wm3, bm3, Wu3, bu3, We1, be1, We2, be2):
    B, N = mu.shape
    E = edge_index.shape[1]
    BN = B * N
    src = edge_index[0]
    dst = edge_index[1]

    mesh = plsc.VectorSubcoreMesh(core_axis_name="c", subcore_axis_name="s",
                                  num_cores=NC, num_subcores=NS)
    sc_params = pltpu.CompilerParams(use_tc_tiling_on_sc=False)
    f = jax.ShapeDtypeStruct

    edge_scratch = [
        pltpu.VMEM((CHUNK,), I32), pltpu.VMEM((CHUNK,), I32),
        pltpu.VMEM((CHUNK // KIDX, KIDX), I32), pltpu.VMEM((CHUNK // KIDX, KIDX), I32),
        pltpu.VMEM((CHUNK, HID), F32), pltpu.VMEM((CHUNK, HID), F32),
    ]
    sc_edge = pl.kernel(
        functools.partial(_sc_edge_body, n=N, e=E, b_total=B),
        out_type=f((BN, HID), F32), mesh=mesh, compiler_params=sc_params,
        scratch_types=edge_scratch + [pltpu.VMEM_SHARED((BN, HID), F32),
                                      pltpu.SemaphoreType.DMA],
    )
    sc_readout = pl.kernel(
        functools.partial(_sc_readout_body, n=N, e=E, b_total=B),
        out_type=f((B * E, HID), F32), mesh=mesh, compiler_params=sc_params,
        scratch_types=edge_scratch + [pltpu.SemaphoreType.DMA],
    )
    sc_assembly = pl.kernel(
        functools.partial(_sc_assembly_body, n=N, e=E, b_total=B),
        out_type=f((B * N * N,), F32), mesh=mesh, compiler_params=sc_params,
        scratch_types=[pltpu.VMEM((CHUNK,), I32), pltpu.VMEM((CHUNK,), I32),
                       pltpu.VMEM((CHUNK,), F32),
                       pltpu.VMEM((CHUNK // KIDX, KIDX), I32),
                       pltpu.VMEM((32768,), F32)],
    )

    tc_prep0 = pl.pallas_call(_tc_prep0_body, out_shape=(f((BN, HID), F32),) * 2)
    tc_rates = pl.pallas_call(
        _tc_rates_body,
        grid=(B * E // 8192,),
        in_specs=[pl.BlockSpec((8192, HID), lambda i: (i, 0)),
                  pl.BlockSpec((1, HID), lambda i: (0, 0)),
                  pl.BlockSpec((1, 1), lambda i: (0, 0))],
        out_specs=pl.BlockSpec((8192, 1), lambda i: (i, 0)),
        out_shape=f((B * E, 1), F32),
    )
    tc_final = pl.pallas_call(
        functools.partial(_tc_final_body, n=N),
        grid=(B,),
        in_specs=[pl.BlockSpec((1, N, N), lambda i: (i, 0, 0))],
        out_specs=pl.BlockSpec((1, N, N), lambda i: (i, 0, 0)),
        out_shape=f((B, N, N), F32),
    )

    t_exp = jnp.broadcast_to(t, (B, N))
    h = jnp.stack([mu, t_exp], axis=-1).reshape(BN, 2)
    A, C = tc_prep0(h, Wm0, bm0.reshape(1, HID))

    layers = [(Wu0, bu0, 2), (Wu1, bu1, HID), (Wu2, bu2, HID), (Wu3, bu3, HID)]
    nxt = [(Wm1, bm1), (Wm2, bm2), (Wm3, bm3), (We1, be1)]
    for (Wu, bu, din), (Wm_n, bm_n) in zip(layers, nxt):
        agg = sc_edge(src, dst, A, C)
        tc_boundary = pl.pallas_call(
            functools.partial(_tc_boundary_body, din=din),
            out_shape=(f((BN, HID), F32),) * 3,
        )
        h, A, C = tc_boundary(h, agg, Wu, bu.reshape(1, HID),
                              Wm_n, bm_n.reshape(1, HID))

    G = sc_readout(src, dst, A, C)
    rates = tc_rates(G, We2.reshape(1, HID), be2.reshape(1, 1))
    rm0 = sc_assembly(src, dst, rates.reshape(B * E))
    return tc_final(rm0.reshape(B, N, N))


# speedup vs baseline: 9.8948x; 1.4512x over previous
import functools

import jax
import jax.numpy as jnp
from jax import lax
from jax.experimental import pallas as pl
from jax.experimental.pallas import tpu as pltpu
from jax.experimental.pallas import tpu_sc as plsc

HID = 64
NC, NS, L = 2, 16, 16
CHUNK = 512
KIDX = 128
F32 = jnp.float32
I32 = jnp.int32
HIGH = lax.Precision.HIGHEST


def _r16(x):
    return x


def _bdot(x, w):
    return jax.lax.dot(x, w, precision=HIGH, preferred_element_type=F32)


def _tc_prep0_body(h_ref, wm_ref, bm_ref, a_ref, c_ref):
    h = h_ref[:]
    wm = wm_ref[:]
    a_ref[:] = (_r16(h[:, 0:1]) * _r16(wm[0:1])
                + _r16(h[:, 1:2]) * _r16(wm[1:2]) + bm_ref[:])
    c_ref[:] = _r16(h[:, 0:1]) * _r16(wm[2:3]) + _r16(h[:, 1:2]) * _r16(wm[3:4])


def _tc_boundary_body(h_ref, agg_ref, wu_ref, bu_ref, wm_ref, bm_ref,
                      h_out, a_out, c_out, *, din):
    wu = wu_ref[:]
    if din == 2:
        h = h_ref[:]
        z = _r16(h[:, 0:1]) * _r16(wu[0:1]) + _r16(h[:, 1:2]) * _r16(wu[1:2])
    else:
        z = _bdot(h_ref[:], wu[:din])
    z = z + _bdot(agg_ref[:], wu[din:]) + bu_ref[:]
    hn = jax.nn.silu(z)
    h_out[:] = hn
    wm = wm_ref[:]
    a_out[:] = _bdot(hn, wm[:HID]) + bm_ref[:]
    c_out[:] = _bdot(hn, wm[HID:])


def _tc_rates_body(g_ref, w_ref, b_ref, r_ref):
    z = jnp.sum(_r16(g_ref[:]) * _r16(w_ref[:]), axis=1, keepdims=True) + b_ref[:]
    r_ref[:] = jnp.logaddexp(z, 0.0)


def _tc_final_body(rm_ref, out_ref, *, n):
    rm = rm_ref[:]
    rs = jnp.sum(rm, axis=2)
    ri = lax.broadcasted_iota(I32, (1, n, n), 1)
    ci = lax.broadcasted_iota(I32, (1, n, n), 2)
    out_ref[:] = jnp.where(ri == ci, -rs[:, :, None], rm)


def _tile_coords(b_per_sc, e_per_tile):
    c = lax.axis_index("c")
    s = lax.axis_index("s")
    tpb = NS // b_per_sc
    lb = s // tpb
    b = c * b_per_sc + lb
    e0 = (s % tpb) * e_per_tile
    return c, s, b, e0


def _build_idx(st_ref, idx_ref, off):
    @plsc.parallel_loop(0, CHUNK // L, unroll=4)
    def _(i):
        idx_ref[pl.ds(i * L, L)] = st_ref[pl.ds(i * L, L)] + off


def _silu_inplace(a_buf, c_buf):
    @plsc.parallel_loop(0, CHUNK, unroll=4)
    def _(r):
        for k in range(HID // L):
            v = a_buf[r, pl.ds(k * L, L)] + c_buf[r, pl.ds(k * L, L)]
            a_buf[r, pl.ds(k * L, L)] = v / (1.0 + jnp.exp(-v))


def _sc_edge_body(src_hbm, dst_hbm, a_hbm, c_hbm, agg_hbm,
                  src_st, dst_st, idxs, idxd, a_buf, c_buf, agg_sh, sem,
                  *, n, e, b_total):
    b_per_sc = b_total // NC
    e_per_tile = (e * b_per_sc) // NS
    c, s, b, e0 = _tile_coords(b_per_sc, e_per_tile)
    rows_per_tile = (b_per_sc * n) // NS

    @plsc.parallel_loop(0, rows_per_tile, unroll=4)
    def _(r):
        for k in range(HID // L):
            a_buf[r, pl.ds(k * L, L)] = jnp.zeros((L,), F32)
    zrow = c * b_per_sc * n + s * rows_per_tile
    pltpu.sync_copy(a_buf.at[pl.ds(0, rows_per_tile)], agg_sh.at[pl.ds(zrow, rows_per_tile)])
    plsc.subcore_barrier()

    def chunk(mc, _):
        base = e0 + mc * CHUNK
        d1 = pltpu.async_copy(src_hbm.at[pl.ds(base, CHUNK)], src_st, sem)
        d2 = pltpu.async_copy(dst_hbm.at[pl.ds(base, CHUNK)], dst_st, sem)
        d1.wait()
        d2.wait()
        _build_idx(src_st, idxs, b * n)
        _build_idx(dst_st, idxd, b * n)
        descs = []
        for j in range(CHUNK // KIDX):
            descs.append(pltpu.async_copy(
                a_hbm.at[idxs.at[pl.ds(j * KIDX, KIDX)]],
                a_buf.at[pl.ds(j * KIDX, KIDX)], sem))
            descs.append(pltpu.async_copy(
                c_hbm.at[idxd.at[pl.ds(j * KIDX, KIDX)]],
                c_buf.at[pl.ds(j * KIDX, KIDX)], sem))
        for d in descs:
            d.wait()
        _silu_inplace(a_buf, c_buf)
        sdescs = []
        for j in range(CHUNK // KIDX):
            sdescs.append(pltpu.async_copy(
                a_buf.at[pl.ds(j * KIDX, KIDX)],
                agg_sh.at[idxd.at[pl.ds(j * KIDX, KIDX)]], sem, add=True))
        for d in sdescs:
            d.wait()
        return 0

    lax.fori_loop(0, e_per_tile // CHUNK, chunk, 0)
    plsc.subcore_barrier()
    pltpu.sync_copy(agg_sh.at[pl.ds(zrow, rows_per_tile)],
                    agg_hbm.at[pl.ds(zrow, rows_per_tile)])


def _sc_readout_body(src_hbm, dst_hbm, p_hbm, q_hbm, g_hbm,
                     src_st, dst_st, idxs, idxd, a_buf, c_buf, sem,
                     *, n, e, b_total):
    b_per_sc = b_total // NC
    e_per_tile = (e * b_per_sc) // NS
    c, s, b, e0 = _tile_coords(b_per_sc, e_per_tile)

    def chunk(mc, _):
        base = e0 + mc * CHUNK
        d1 = pltpu.async_copy(src_hbm.at[pl.ds(base, CHUNK)], src_st, sem)
        d2 = pltpu.async_copy(dst_hbm.at[pl.ds(base, CHUNK)], dst_st, sem)
        d1.wait()
        d2.wait()
        _build_idx(src_st, idxs, b * n)
        _build_idx(dst_st, idxd, b * n)
        descs = []
        for j in range(CHUNK // KIDX):
            descs.append(pltpu.async_copy(
                p_hbm.at[idxs.at[pl.ds(j * KIDX, KIDX)]],
                a_buf.at[pl.ds(j * KIDX, KIDX)], sem))
            descs.append(pltpu.async_copy(
                q_hbm.at[idxd.at[pl.ds(j * KIDX, KIDX)]],
                c_buf.at[pl.ds(j * KIDX, KIDX)], sem))
        for d in descs:
            d.wait()
        _silu_inplace(a_buf, c_buf)
        pltpu.sync_copy(a_buf, g_hbm.at[pl.ds(b * e + base, CHUNK)])
        return 0

    lax.fori_loop(0, e_per_tile // CHUNK, chunk, 0)


def _sc_assembly_body(src_hbm, dst_hbm, rate_hbm, rm_hbm,
                      src_st, dst_st, rate_st, idx, zbuf, sem,
                      *, n, e, b_total):
    b_per_sc = b_total // NC
    e_per_tile = (e * b_per_sc) // NS
    c, s, b, e0 = _tile_coords(b_per_sc, e_per_tile)

    zwords = zbuf.shape[0]
    @plsc.parallel_loop(0, zwords // L, unroll=4)
    def _(i):
        zbuf[pl.ds(i * L, L)] = jnp.zeros((L,), F32)
    words_per_tile = (b_per_sc * n * n) // NS
    zstart = c * b_per_sc * n * n + s * words_per_tile
    zdescs = [pltpu.async_copy(zbuf, rm_hbm.at[pl.ds(zstart + z * zwords, zwords)],
                               sem)
              for z in range(words_per_tile // zwords)]
    for d in zdescs:
        d.wait()
    plsc.subcore_barrier()

    def chunk(mc, _):
        base = e0 + mc * CHUNK
        d1 = pltpu.async_copy(src_hbm.at[pl.ds(base, CHUNK)], src_st, sem)
        d2 = pltpu.async_copy(dst_hbm.at[pl.ds(base, CHUNK)], dst_st, sem)
        d3 = pltpu.async_copy(rate_hbm.at[pl.ds(b * e + base, CHUNK)], rate_st, sem)
        d1.wait()
        d2.wait()
        d3.wait()
        @plsc.parallel_loop(0, CHUNK // L, unroll=4)
        def _(i):
            vs = src_st[pl.ds(i * L, L)]
            vd = dst_st[pl.ds(i * L, L)]
            idx[pl.ds(i * L, L)] = vs * n + vd + b * n * n
        sdescs = []
        for j in range(CHUNK // KIDX):
            sdescs.append(pltpu.async_copy(
                rate_st.at[pl.ds(j * KIDX, KIDX)],
                rm_hbm.at[idx.at[pl.ds(j * KIDX, KIDX)]], sem))
        for d in sdescs:
            d.wait()
        return 0

    lax.fori_loop(0, e_per_tile // CHUNK, chunk, 0)


def kernel(mu, t, edge_index, Wm0, bm0, Wu0, bu0, Wm1, bm1, Wu1, bu1,
           Wm2, bm2, Wu2, bu2, Wm3, bm3, Wu3, bu3, We1, be1, We2, be2):
    B, N = mu.shape
    E = edge_index.shape[1]
    BN = B * N
    src = edge_index[0]
    dst = edge_index[1]

    mesh = plsc.VectorSubcoreMesh(core_axis_name="c", subcore_axis_name="s",
                                  num_cores=NC, num_subcores=NS)
    sc_params = pltpu.CompilerParams(use_tc_tiling_on_sc=False)
    f = jax.ShapeDtypeStruct

    edge_scratch = [
        pltpu.VMEM((CHUNK,), I32), pltpu.VMEM((CHUNK,), I32),
        pltpu.VMEM((CHUNK,), I32), pltpu.VMEM((CHUNK,), I32),
        pltpu.VMEM((CHUNK, HID), F32), pltpu.VMEM((CHUNK, HID), F32),
    ]
    sc_edge = pl.kernel(
        functools.partial(_sc_edge_body, n=N, e=E, b_total=B),
        out_type=f((BN, HID), F32), mesh=mesh, compiler_params=sc_params,
        scratch_types=edge_scratch + [pltpu.VMEM_SHARED((BN, HID), F32),
                                      pltpu.SemaphoreType.DMA],
    )
    sc_readout = pl.kernel(
        functools.partial(_sc_readout_body, n=N, e=E, b_total=B),
        out_type=f((B * E, HID), F32), mesh=mesh, compiler_params=sc_params,
        scratch_types=edge_scratch + [pltpu.SemaphoreType.DMA],
    )
    sc_assembly = pl.kernel(
        functools.partial(_sc_assembly_body, n=N, e=E, b_total=B),
        out_type=f((B * N * N,), F32), mesh=mesh, compiler_params=sc_params,
        scratch_types=[pltpu.VMEM((CHUNK,), I32), pltpu.VMEM((CHUNK,), I32),
                       pltpu.VMEM((CHUNK,), F32),
                       pltpu.VMEM((CHUNK,), I32),
                       pltpu.VMEM((32768,), F32),
                       pltpu.SemaphoreType.DMA],
    )

    tc_prep0 = pl.pallas_call(_tc_prep0_body, out_shape=(f((BN, HID), F32),) * 2)
    tc_rates = pl.pallas_call(
        _tc_rates_body,
        grid=(B * E // 8192,),
        in_specs=[pl.BlockSpec((8192, HID), lambda i: (i, 0)),
                  pl.BlockSpec((1, HID), lambda i: (0, 0)),
                  pl.BlockSpec((1, 1), lambda i: (0, 0))],
        out_specs=pl.BlockSpec((8192, 1), lambda i: (i, 0)),
        out_shape=f((B * E, 1), F32),
    )
    tc_final = pl.pallas_call(
        functools.partial(_tc_final_body, n=N),
        grid=(B,),
        in_specs=[pl.BlockSpec((1, N, N), lambda i: (i, 0, 0))],
        out_specs=pl.BlockSpec((1, N, N), lambda i: (i, 0, 0)),
        out_shape=f((B, N, N), F32),
    )

    t_exp = jnp.broadcast_to(t, (B, N))
    h = jnp.stack([mu, t_exp], axis=-1).reshape(BN, 2)
    A, C = tc_prep0(h, Wm0, bm0.reshape(1, HID))

    layers = [(Wu0, bu0, 2), (Wu1, bu1, HID), (Wu2, bu2, HID), (Wu3, bu3, HID)]
    nxt = [(Wm1, bm1), (Wm2, bm2), (Wm3, bm3), (We1, be1)]
    for (Wu, bu, din), (Wm_n, bm_n) in zip(layers, nxt):
        agg = sc_edge(src, dst, A, C)
        tc_boundary = pl.pallas_call(
            functools.partial(_tc_boundary_body, din=din),
            out_shape=(f((BN, HID), F32),) * 3,
        )
        h, A, C = tc_boundary(h, agg, Wu, bu.reshape(1, HID),
                              Wm_n, bm_n.reshape(1, HID))

    G = sc_readout(src, dst, A, C)
    rates = tc_rates(G, We2.reshape(1, HID), be2.reshape(1, 1))
    rm0 = sc_assembly(src, dst, rates.reshape(B * E))
    return tc_final(rm0.reshape(B, N, N))
